# Initial kernel scaffold; baseline (speedup 1.0000x reference)
import jax
import jax.numpy as jnp
from jax.experimental import pallas as pl


def kernel(inputs, embeddings):
    raise NotImplementedError("write your pallas kernel here")


# speedup vs baseline: 1.4171x; 1.4171x over previous
import functools

import jax
import jax.numpy as jnp
from jax import lax
from jax.experimental import pallas as pl
from jax.experimental.pallas import tpu as pltpu
from jax.experimental.pallas import tpu_sc as plsc

_BETA = 0.25
_TILE_M = 512


_BLK = 4096


def _dist_argmin_kernel(x_ref, embt_ref, idx_ref, msum_ref):
    x = x_ref[...]
    embt = embt_ref[...]
    e2 = jnp.sum(embt * embt, axis=0, keepdims=True)
    x2 = jnp.sum(x * x, axis=1, keepdims=True)
    mm = jnp.dot(x.astype(jnp.bfloat16), embt.astype(jnp.bfloat16),
                 preferred_element_type=jnp.float32)
    d = x2 + e2 - 2.0 * mm

    iota = lax.broadcasted_iota(jnp.int32, (d.shape[0], _BLK), 1)
    d0 = d[:, :_BLK]
    d1 = d[:, _BLK:]
    mind0 = jnp.min(d0, axis=1, keepdims=True)
    mind1 = jnp.min(d1, axis=1, keepdims=True)
    idx0 = jnp.min(jnp.where(d0 == mind0, iota, _BLK), axis=1)
    idx1 = jnp.min(jnp.where(d1 == mind1, iota, _BLK), axis=1) + _BLK
    mind0_bf = mind0.astype(jnp.bfloat16).astype(jnp.float32)
    accept1 = mind1 < mind0_bf
    idx_ref[...] = jnp.where(accept1[:, 0], idx1, idx0)
    dsel = jnp.where(accept1, mind1, mind0)

    @pl.when(pl.program_id(0) == 0)
    def _init():
        msum_ref[...] = jnp.zeros_like(msum_ref)

    msum_ref[...] += jnp.sum(dsel, axis=0, keepdims=True)


@functools.partial(jax.jit, static_argnums=())
def _dist_argmin(xf, embt):
    m, d = xf.shape
    ne = embt.shape[1]
    grid = (m // _TILE_M,)
    return pl.pallas_call(
        _dist_argmin_kernel,
        grid=grid,
        in_specs=[
            pl.BlockSpec((_TILE_M, d), lambda i: (i, 0)),
            pl.BlockSpec((d, ne), lambda i: (0, 0)),
        ],
        out_specs=[
            pl.BlockSpec((_TILE_M,), lambda i: (i,)),
            pl.BlockSpec((1, 1), lambda i: (0, 0)),
        ],
        out_shape=[
            jax.ShapeDtypeStruct((m,), jnp.int32),
            jax.ShapeDtypeStruct((1, 1), jnp.float32),
        ],
    )(xf, embt)


def _make_sc_gather(num_rows, d):
    info = plsc.get_sparse_core_info()
    nw = info.num_cores * info.num_subcores
    rows_per_w = num_rows // nw
    nc = info.num_cores
    mesh = plsc.VectorSubcoreMesh(core_axis_name="c", subcore_axis_name="s")

    @functools.partial(
        pl.kernel,
        mesh=mesh,
        out_type=jax.ShapeDtypeStruct((num_rows, d), jnp.float32),
        scratch_types=[
            pltpu.VMEM((rows_per_w,), jnp.int32),
            pltpu.VMEM((rows_per_w, d), jnp.float32),
            pltpu.SemaphoreType.DMA,
        ],
        compiler_params=pltpu.CompilerParams(use_tc_tiling_on_sc=False),
    )
    def gather(table_hbm, idx_hbm, out_hbm, idx_v, rows_v, sem):
        wid = lax.axis_index("s") * nc + lax.axis_index("c")
        base = wid * rows_per_w
        pltpu.sync_copy(idx_hbm.at[pl.ds(base, rows_per_w)], idx_v)
        pltpu.async_copy(table_hbm.at[idx_v], rows_v, sem).wait()
        pltpu.sync_copy(rows_v, out_hbm.at[pl.ds(base, rows_per_w)])

    return gather


def kernel(inputs, embeddings):
    b, c, h, w = inputs.shape
    m = b * h * w
    xf = jnp.transpose(inputs, (0, 2, 3, 1)).reshape(m, c)
    idx, msum = _dist_argmin(xf, embeddings.T)
    q = _make_sc_gather(m, c)(embeddings, idx)
    quantized = jnp.transpose(q.reshape(b, h, w, c), (0, 3, 1, 2))
    loss = (1.0 + _BETA) * (msum[0, 0] / (m * c))
    return quantized, loss


# speedup vs baseline: 1.4226x; 1.0039x over previous
import functools

import jax
import jax.numpy as jnp
from jax import lax
from jax.experimental import pallas as pl
from jax.experimental.pallas import tpu as pltpu
from jax.experimental.pallas import tpu_sc as plsc

_BETA = 0.25
_TILE_M = 512


_BLK = 4096


def _dist_argmin_kernel(x_ref, embt_ref, idx_ref, msum_ref, ebf_ref, e2_ref):
    @pl.when(pl.program_id(0) == 0)
    def _precompute():
        embt0 = embt_ref[...]
        ebf_ref[...] = embt0.astype(jnp.bfloat16)
        e2_ref[...] = jnp.sum(embt0 * embt0, axis=0, keepdims=True)

    x = x_ref[...]
    e2 = e2_ref[...]
    x2 = jnp.sum(x * x, axis=1, keepdims=True)
    mm = jnp.dot(x.astype(jnp.bfloat16), ebf_ref[...],
                 preferred_element_type=jnp.float32)
    d = x2 + e2 - 2.0 * mm

    iota = lax.broadcasted_iota(jnp.int32, (d.shape[0], _BLK), 1)
    d0 = d[:, :_BLK]
    d1 = d[:, _BLK:]
    mind0 = jnp.min(d0, axis=1, keepdims=True)
    mind1 = jnp.min(d1, axis=1, keepdims=True)
    idx0 = jnp.min(jnp.where(d0 == mind0, iota, _BLK), axis=1)
    idx1 = jnp.min(jnp.where(d1 == mind1, iota, _BLK), axis=1) + _BLK
    mind0_bf = mind0.astype(jnp.bfloat16).astype(jnp.float32)
    accept1 = mind1 < mind0_bf
    idx_ref[...] = jnp.where(accept1[:, 0], idx1, idx0)
    dsel = jnp.where(accept1, mind1, mind0)

    @pl.when(pl.program_id(0) == 0)
    def _init():
        msum_ref[...] = jnp.zeros_like(msum_ref)

    msum_ref[...] += jnp.sum(dsel, axis=0, keepdims=True)


@functools.partial(jax.jit, static_argnums=())
def _dist_argmin(xf, embt):
    m, d = xf.shape
    ne = embt.shape[1]
    grid = (m // _TILE_M,)
    return pl.pallas_call(
        _dist_argmin_kernel,
        grid=grid,
        in_specs=[
            pl.BlockSpec((_TILE_M, d), lambda i: (i, 0)),
            pl.BlockSpec((d, ne), lambda i: (0, 0)),
        ],
        out_specs=[
            pl.BlockSpec((_TILE_M,), lambda i: (i,)),
            pl.BlockSpec((1, 1), lambda i: (0, 0)),
        ],
        out_shape=[
            jax.ShapeDtypeStruct((m,), jnp.int32),
            jax.ShapeDtypeStruct((1, 1), jnp.float32),
        ],
        scratch_shapes=[
            pltpu.VMEM((d, ne), jnp.bfloat16),
            pltpu.VMEM((1, ne), jnp.float32),
        ],
    )(xf, embt)


def _make_sc_gather(num_rows, d):
    info = plsc.get_sparse_core_info()
    nw = info.num_cores * info.num_subcores
    rows_per_w = num_rows // nw
    nc = info.num_cores
    mesh = plsc.VectorSubcoreMesh(core_axis_name="c", subcore_axis_name="s")

    @functools.partial(
        pl.kernel,
        mesh=mesh,
        out_type=jax.ShapeDtypeStruct((num_rows, d), jnp.float32),
        scratch_types=[
            pltpu.VMEM((rows_per_w,), jnp.int32),
            pltpu.VMEM((rows_per_w, d), jnp.float32),
            pltpu.SemaphoreType.DMA,
        ],
        compiler_params=pltpu.CompilerParams(use_tc_tiling_on_sc=False),
    )
    def gather(table_hbm, idx_hbm, out_hbm, idx_v, rows_v, sem):
        wid = lax.axis_index("s") * nc + lax.axis_index("c")
        base = wid * rows_per_w
        pltpu.sync_copy(idx_hbm.at[pl.ds(base, rows_per_w)], idx_v)
        pltpu.async_copy(table_hbm.at[idx_v], rows_v, sem).wait()
        pltpu.sync_copy(rows_v, out_hbm.at[pl.ds(base, rows_per_w)])

    return gather


def kernel(inputs, embeddings):
    b, c, h, w = inputs.shape
    m = b * h * w
    xf = jnp.transpose(inputs, (0, 2, 3, 1)).reshape(m, c)
    idx, msum = _dist_argmin(xf, embeddings.T)
    q = _make_sc_gather(m, c)(embeddings, idx)
    quantized = jnp.transpose(q.reshape(b, h, w, c), (0, 3, 1, 2))
    loss = (1.0 + _BETA) * (msum[0, 0] / (m * c))
    return quantized, loss
